# baseline (speedup 1.0000x reference)
import functools

import jax
import jax.numpy as jnp
from jax import lax
from jax.experimental import pallas as pl
from jax.experimental.pallas import tpu as pltpu
from jax.experimental.pallas import tpu_sc as plsc

_N = 10000
_E = 320000
_D = 128
_H = 128
_O = 64

_NC = 2
_NS = 16
_NW = _NC * _NS
_NP = 10240
_RP = _NP // _NS

_C1, _K1 = 104, 100
_C2, _K2 = 128, 80
_EP1 = _NW * _K1 * _C1
_EP2 = _NW * _K2 * _C2


def _make_sc_pass(width, cchunk, kchunks, with_deg):
  mesh = plsc.VectorSubcoreMesh(core_axis_name="c", subcore_axis_name="s")
  out_type = [jax.ShapeDtypeStruct((_NC, _NP, width), jnp.float32)]
  scratch = [
      pltpu.VMEM((kchunks, cchunk), jnp.int32),
      pltpu.VMEM((kchunks, cchunk), jnp.int32),
      pltpu.VMEM((cchunk, width), jnp.float32),
      pltpu.VMEM((cchunk, width), jnp.float32),
      pltpu.SemaphoreType.DMA,
      pltpu.SemaphoreType.DMA,
      pltpu.SemaphoreType.DMA,
      pltpu.SemaphoreType.DMA,
      pltpu.VMEM_SHARED((_NP, width), jnp.float32),
  ]
  if with_deg:
    out_type.append(jax.ShapeDtypeStruct((_NC, _NP), jnp.float32))
    scratch += [
        pltpu.VMEM((cchunk,), jnp.float32),
        pltpu.VMEM((_RP,), jnp.float32),
        pltpu.VMEM_SHARED((_NP,), jnp.float32),
        pltpu.SemaphoreType.DMA,
        pltpu.SemaphoreType.DMA,
    ]

  def body(table, src_hbm, dst_hbm, *refs):
    out_agg = refs[0]
    refs = refs[1:]
    if with_deg:
      out_deg, refs = refs[0], refs[1:]
    (src_v, dst_v, rows0, rows1, gsem0, gsem1, ssem0, ssem1,
     acc_sh) = refs[:9]
    if with_deg:
      ones_v, zdeg_v, deg_sh, dsem0, dsem1 = refs[9:14]
    c = lax.axis_index("c")
    s = lax.axis_index("s")
    wid = c * _NS + s
    base = s * _RP

    npack = width // 16
    rows = (rows0, rows1)

    for b in range(2):
      def zrow_b(t, carry, _b=b):
        rows[_b][t // npack, pl.ds((t % npack) * 16, 16)] = jnp.zeros(
            (16,), jnp.float32)
        return carry
      lax.fori_loop(0, cchunk * npack, zrow_b, 0)

    nfull = _RP // cchunk
    rem = _RP - nfull * cchunk
    for k in range(nfull):
      pltpu.sync_copy(rows[0], acc_sh.at[pl.ds(base + k * cchunk, cchunk)])
    if rem:
      pltpu.sync_copy(rows[0].at[pl.ds(0, rem)],
                      acc_sh.at[pl.ds(base + nfull * cchunk, rem)])

    if with_deg:
      def zdeg(t, carry):
        zdeg_v[pl.ds(t * 16, 16)] = jnp.zeros((16,), jnp.float32)
        return carry

      lax.fori_loop(0, _RP // 16, zdeg, 0)
      pltpu.sync_copy(zdeg_v, deg_sh.at[pl.ds(base, _RP)])

      def ones(t, carry):
        ones_v[pl.ds(t * 16, 16)] = jnp.ones((16,), jnp.float32)
        return carry

      lax.fori_loop(0, cchunk // 16, ones, 0)

    pltpu.sync_copy(src_hbm.at[wid], src_v)
    pltpu.sync_copy(dst_hbm.at[wid], dst_v)

    plsc.subcore_barrier()

    def g_issue(j, b):
      pltpu.async_copy(table.at[src_v.at[j]], rows[b], (gsem0, gsem1)[b])

    def g_wait(b):
      pltpu.make_async_copy(table.at[src_v.at[0]], rows[b],
                            (gsem0, gsem1)[b]).wait()

    def s_issue(j, b):
      pltpu.async_copy(rows[b], acc_sh.at[dst_v.at[j]],
                       (ssem0, ssem1)[b], add=True)
      if with_deg:
        pltpu.async_copy(ones_v, deg_sh.at[dst_v.at[j]],
                         (dsem0, dsem1)[b], add=True)

    def s_wait(b):
      pltpu.make_async_copy(rows[b], acc_sh.at[dst_v.at[0]],
                            (ssem0, ssem1)[b]).wait()
      if with_deg:
        pltpu.make_async_copy(ones_v, deg_sh.at[dst_v.at[0]],
                              (dsem0, dsem1)[b]).wait()

    pltpu.async_copy(rows1, acc_sh.at[dst_v.at[0]], ssem1, add=True)
    if with_deg:
      pltpu.async_copy(zdeg_v.at[pl.ds(0, cchunk)],
                       deg_sh.at[dst_v.at[0]], dsem1, add=True)
    g_issue(0, 0)

    def pair(p, carry):
      j = 2 * p
      s_wait(1)
      g_issue(j + 1, 1)
      g_wait(0)
      s_issue(j, 0)
      s_wait(0)
      g_issue(jnp.minimum(j + 2, kchunks - 1), 0)
      g_wait(1)
      s_issue(j + 1, 1)
      return carry

    lax.fori_loop(0, kchunks // 2, pair, 0)

    s_wait(1)
    g_wait(0)

    plsc.subcore_barrier()

    for k in range(_RP // 128):
      sl = pl.ds(base + k * 128, 128)
      pltpu.sync_copy(acc_sh.at[sl], out_agg.at[c, sl])
    if with_deg:
      pltpu.sync_copy(deg_sh.at[pl.ds(base, _RP)],
                      out_deg.at[c, pl.ds(base, _RP)])

  return pl.kernel(body, out_type=tuple(out_type), mesh=mesh,
                   scratch_types=scratch,
                   compiler_params=pltpu.CompilerParams(
                       use_tc_tiling_on_sc=False))


_sc_pass1 = _make_sc_pass(_D, _C1, _K1, True)
_sc_pass2 = _make_sc_pass(_O, _C2, _K2, False)

_BR = 1024
_GRID = _NP // _BR


def _tc_a_body(x_ref, ws1, wn1, sb, cb, agg, degt, wn2, h_ref, hw2_ref):
  d = jnp.maximum(degt[:, 0:1] + degt[:, 1:2], 1.0)
  hn = (agg[0] + agg[1]) / d
  hl = jnp.dot(x_ref[...], ws1[...], preferred_element_type=jnp.float32)
  hl = hl + jnp.dot(hn, wn1[...], preferred_element_type=jnp.float32)
  h = jnp.maximum(hl * sb[...] + cb[...], 0.0)
  h_ref[...] = h
  hw2_ref[...] = jnp.dot(h, wn2[...], preferred_element_type=jnp.float32)


_tc_a = pl.pallas_call(
    _tc_a_body,
    grid=(_GRID,),
    in_specs=[
        pl.BlockSpec((_BR, _D), lambda i: (i, 0)),
        pl.BlockSpec((_D, _H), lambda i: (0, 0)),
        pl.BlockSpec((_D, _H), lambda i: (0, 0)),
        pl.BlockSpec((1, _H), lambda i: (0, 0)),
        pl.BlockSpec((1, _H), lambda i: (0, 0)),
        pl.BlockSpec((_NC, _BR, _D), lambda i: (0, i, 0)),
        pl.BlockSpec((_BR, _NC), lambda i: (i, 0)),
        pl.BlockSpec((_H, _O), lambda i: (0, 0)),
    ],
    out_specs=[
        pl.BlockSpec((_BR, _H), lambda i: (i, 0)),
        pl.BlockSpec((_BR, _O), lambda i: (i, 0)),
    ],
    out_shape=[
        jax.ShapeDtypeStruct((_N, _H), jnp.float32),
        jax.ShapeDtypeStruct((_N, _O), jnp.float32),
    ],
)


def _tc_b_body(h_ref, ws2, agg2, degt, b2, out_ref):
  d = jnp.maximum(degt[:, 0:1] + degt[:, 1:2], 1.0)
  hn2 = (agg2[0] + agg2[1]) / d
  out_ref[...] = (
      jnp.dot(h_ref[...], ws2[...], preferred_element_type=jnp.float32)
      + hn2 + b2[...])


_tc_b = pl.pallas_call(
    _tc_b_body,
    grid=(_GRID,),
    in_specs=[
        pl.BlockSpec((_BR, _H), lambda i: (i, 0)),
        pl.BlockSpec((_H, _O), lambda i: (0, 0)),
        pl.BlockSpec((_NC, _BR, _O), lambda i: (0, i, 0)),
        pl.BlockSpec((_BR, _NC), lambda i: (i, 0)),
        pl.BlockSpec((1, _O), lambda i: (0, 0)),
    ],
    out_specs=pl.BlockSpec((_BR, _O), lambda i: (i, 0)),
    out_shape=jax.ShapeDtypeStruct((_N, _O), jnp.float32),
)


def kernel(x, edge_index, W_self1, W_neigh1, b1, gamma1, beta1,
           W_self2, W_neigh2, b2):
  def pad_edges(ep, k, cc):
    pad = ep - _E
    srcp = jnp.concatenate([edge_index[0], jnp.zeros((pad,), jnp.int32)])
    pad_dst = _N + jnp.arange(pad, dtype=jnp.int32) % (_NP - _N)
    dstp = jnp.concatenate([edge_index[1], pad_dst])
    return srcp.reshape(_NW, k, cc), dstp.reshape(_NW, k, cc)

  s1, d1 = pad_edges(_EP1, _K1, _C1)
  s2, d2 = pad_edges(_EP2, _K2, _C2)

  aggx, deg = _sc_pass1(x, s1, d1)
  degt = deg.T

  sb = (gamma1 * (1.0 / jnp.sqrt(1.0 + 1e-5))).reshape(1, _H)
  cb = (b1 * sb[0] + beta1).reshape(1, _H)

  h, hw2 = _tc_a(x, W_self1, W_neigh1, sb, cb, aggx, degt, W_neigh2)
  (agg2,) = _sc_pass2(hw2, s2, d2)
  out = _tc_b(h, W_self2, agg2, degt, b2.reshape(1, _O))
  return out


# speedup vs baseline: 1.0063x; 1.0063x over previous
import functools

import jax
import jax.numpy as jnp
from jax import lax
from jax.experimental import pallas as pl
from jax.experimental.pallas import tpu as pltpu
from jax.experimental.pallas import tpu_sc as plsc

_N = 10000
_E = 320000
_D = 128
_H = 128
_O = 64

_NC = 2
_NS = 16
_NW = _NC * _NS
_NP = 10240
_RP = _NP // _NS

_C1, _K1, _G1 = 96, 108, 2
_C2, _K2, _G2 = 128, 80, 4
_EP1 = _NW * _K1 * _C1
_EP2 = _NW * _K2 * _C2


def _make_sc_pass(width, cchunk, kchunks, grp, with_deg):
  mesh = plsc.VectorSubcoreMesh(core_axis_name="c", subcore_axis_name="s")
  out_type = [jax.ShapeDtypeStruct((_NC, _NP, width), jnp.float32)]
  scratch = [
      pltpu.VMEM((kchunks, cchunk), jnp.int32),
      pltpu.VMEM((kchunks, cchunk), jnp.int32),
  ]
  scratch += [pltpu.VMEM((cchunk, width), jnp.float32)
              for _ in range(grp)]
  scratch += [pltpu.SemaphoreType.DMA for _ in range(2 * grp)]
  scratch.append(pltpu.VMEM_SHARED((_NP, width), jnp.float32))
  if with_deg:
    out_type.append(jax.ShapeDtypeStruct((_NC, _NP), jnp.float32))
    scratch += [
        pltpu.VMEM((cchunk,), jnp.float32),
        pltpu.VMEM((_RP,), jnp.float32),
        pltpu.VMEM_SHARED((_NP,), jnp.float32),
    ]
    scratch += [pltpu.SemaphoreType.DMA for _ in range(grp)]

  def body(table, src_hbm, dst_hbm, *refs):
    out_agg = refs[0]
    refs = refs[1:]
    if with_deg:
      out_deg, refs = refs[0], refs[1:]
    src_v, dst_v = refs[0], refs[1]
    rows = refs[2:2 + grp]
    gsem = refs[2 + grp:2 + 2 * grp]
    ssem = refs[2 + 2 * grp:2 + 3 * grp]
    acc_sh = refs[2 + 3 * grp]
    if with_deg:
      ones_v, zdeg_v, deg_sh = refs[3 + 3 * grp:6 + 3 * grp]
      dsem = refs[6 + 3 * grp:6 + 4 * grp]
    c = lax.axis_index("c")
    s = lax.axis_index("s")
    wid = c * _NS + s
    base = s * _RP

    npack = width // 16

    def zrow(t, carry):
      rows[0][t // npack, pl.ds((t % npack) * 16, 16)] = jnp.zeros(
          (16,), jnp.float32)
      return carry

    lax.fori_loop(0, cchunk * npack, zrow, 0)

    nfull = _RP // cchunk
    rem = _RP - nfull * cchunk
    for k in range(nfull):
      pltpu.sync_copy(rows[0], acc_sh.at[pl.ds(base + k * cchunk, cchunk)])
    if rem:
      pltpu.sync_copy(rows[0].at[pl.ds(0, rem)],
                      acc_sh.at[pl.ds(base + nfull * cchunk, rem)])

    if with_deg:
      def zdeg(t, carry):
        zdeg_v[pl.ds(t * 16, 16)] = jnp.zeros((16,), jnp.float32)
        return carry

      lax.fori_loop(0, _RP // 16, zdeg, 0)
      pltpu.sync_copy(zdeg_v, deg_sh.at[pl.ds(base, _RP)])

      def ones(t, carry):
        ones_v[pl.ds(t * 16, 16)] = jnp.ones((16,), jnp.float32)
        return carry

      lax.fori_loop(0, cchunk // 16, ones, 0)

    pltpu.sync_copy(src_hbm.at[wid], src_v)
    pltpu.sync_copy(dst_hbm.at[wid], dst_v)

    plsc.subcore_barrier()

    def group(g, carry):
      ch0 = g * grp
      gd = [pltpu.async_copy(table.at[src_v.at[ch0 + t]], rows[t],
                             gsem[t]) for t in range(grp)]
      sd = []
      dd = []
      for t in range(grp):
        gd[t].wait()
        sd.append(pltpu.async_copy(rows[t], acc_sh.at[dst_v.at[ch0 + t]],
                                   ssem[t], add=True))
        if with_deg:
          dd.append(pltpu.async_copy(ones_v, deg_sh.at[dst_v.at[ch0 + t]],
                                     dsem[t], add=True))
      for t in range(grp):
        sd[t].wait()
        if with_deg:
          dd[t].wait()
      return carry

    lax.fori_loop(0, kchunks // grp, group, 0)

    plsc.subcore_barrier()

    for k in range(_RP // 128):
      sl = pl.ds(base + k * 128, 128)
      pltpu.sync_copy(acc_sh.at[sl], out_agg.at[c, sl])
    if with_deg:
      pltpu.sync_copy(deg_sh.at[pl.ds(base, _RP)],
                      out_deg.at[c, pl.ds(base, _RP)])

  return pl.kernel(body, out_type=tuple(out_type), mesh=mesh,
                   scratch_types=scratch,
                   compiler_params=pltpu.CompilerParams(
                       use_tc_tiling_on_sc=False))


_sc_pass1 = _make_sc_pass(_D, _C1, _K1, _G1, True)
_sc_pass2 = _make_sc_pass(_O, _C2, _K2, _G2, False)

_BR = 1024
_GRID = _NP // _BR


def _tc_a_body(x_ref, ws1, wn1, sb, cb, agg, degt, wn2, h_ref, hw2_ref):
  d = jnp.maximum(degt[:, 0:1] + degt[:, 1:2], 1.0)
  hn = (agg[0] + agg[1]) / d
  hl = jnp.dot(x_ref[...], ws1[...], preferred_element_type=jnp.float32)
  hl = hl + jnp.dot(hn, wn1[...], preferred_element_type=jnp.float32)
  h = jnp.maximum(hl * sb[...] + cb[...], 0.0)
  h_ref[...] = h
  hw2_ref[...] = jnp.dot(h, wn2[...], preferred_element_type=jnp.float32)


_tc_a = pl.pallas_call(
    _tc_a_body,
    grid=(_GRID,),
    in_specs=[
        pl.BlockSpec((_BR, _D), lambda i: (i, 0)),
        pl.BlockSpec((_D, _H), lambda i: (0, 0)),
        pl.BlockSpec((_D, _H), lambda i: (0, 0)),
        pl.BlockSpec((1, _H), lambda i: (0, 0)),
        pl.BlockSpec((1, _H), lambda i: (0, 0)),
        pl.BlockSpec((_NC, _BR, _D), lambda i: (0, i, 0)),
        pl.BlockSpec((_BR, _NC), lambda i: (i, 0)),
        pl.BlockSpec((_H, _O), lambda i: (0, 0)),
    ],
    out_specs=[
        pl.BlockSpec((_BR, _H), lambda i: (i, 0)),
        pl.BlockSpec((_BR, _O), lambda i: (i, 0)),
    ],
    out_shape=[
        jax.ShapeDtypeStruct((_N, _H), jnp.float32),
        jax.ShapeDtypeStruct((_N, _O), jnp.float32),
    ],
)


def _tc_b_body(h_ref, ws2, agg2, degt, b2, out_ref):
  d = jnp.maximum(degt[:, 0:1] + degt[:, 1:2], 1.0)
  hn2 = (agg2[0] + agg2[1]) / d
  out_ref[...] = (
      jnp.dot(h_ref[...], ws2[...], preferred_element_type=jnp.float32)
      + hn2 + b2[...])


_tc_b = pl.pallas_call(
    _tc_b_body,
    grid=(_GRID,),
    in_specs=[
        pl.BlockSpec((_BR, _H), lambda i: (i, 0)),
        pl.BlockSpec((_H, _O), lambda i: (0, 0)),
        pl.BlockSpec((_NC, _BR, _O), lambda i: (0, i, 0)),
        pl.BlockSpec((_BR, _NC), lambda i: (i, 0)),
        pl.BlockSpec((1, _O), lambda i: (0, 0)),
    ],
    out_specs=pl.BlockSpec((_BR, _O), lambda i: (i, 0)),
    out_shape=jax.ShapeDtypeStruct((_N, _O), jnp.float32),
)


def kernel(x, edge_index, W_self1, W_neigh1, b1, gamma1, beta1,
           W_self2, W_neigh2, b2):
  def pad_edges(ep, k, cc):
    pad = ep - _E
    srcp = jnp.concatenate([edge_index[0], jnp.zeros((pad,), jnp.int32)])
    pad_dst = _N + jnp.arange(pad, dtype=jnp.int32) % (_NP - _N)
    dstp = jnp.concatenate([edge_index[1], pad_dst])
    return srcp.reshape(_NW, k, cc), dstp.reshape(_NW, k, cc)

  s1, d1 = pad_edges(_EP1, _K1, _C1)
  s2, d2 = pad_edges(_EP2, _K2, _C2)

  aggx, deg = _sc_pass1(x, s1, d1)
  degt = deg.T

  sb = (gamma1 * (1.0 / jnp.sqrt(1.0 + 1e-5))).reshape(1, _H)
  cb = (b1 * sb[0] + beta1).reshape(1, _H)

  h, hw2 = _tc_a(x, W_self1, W_neigh1, sb, cb, aggx, degt, W_neigh2)
  (agg2,) = _sc_pass2(hw2, s2, d2)
  out = _tc_b(h, W_self2, agg2, degt, b2.reshape(1, _O))
  return out


# speedup vs baseline: 2.4224x; 2.4072x over previous
import functools

import jax
import jax.numpy as jnp
from jax import lax
from jax.experimental import pallas as pl
from jax.experimental.pallas import tpu as pltpu
from jax.experimental.pallas import tpu_sc as plsc

_N = 10000
_E = 320000
_D = 128
_H = 128
_O = 64

_NC = 2
_NS = 16
_NW = _NC * _NS
_NP = 10240
_RP = _NP // _NS

_C1, _K1, _G1 = 80, 125, 1
_C2, _K2, _G2 = 80, 125, 1
_EP1 = _NW * _K1 * _C1
_EP2 = _NW * _K2 * _C2


def _make_sc_pass(width, cchunk, kchunks, grp, with_deg):
  mesh = plsc.VectorSubcoreMesh(core_axis_name="c", subcore_axis_name="s")
  out_type = [jax.ShapeDtypeStruct((_NC, _NP, width), jnp.float32)]
  scratch = [
      pltpu.VMEM((kchunks, cchunk), jnp.int32),
      pltpu.VMEM((kchunks, cchunk), jnp.int32),
  ]
  scratch += [pltpu.VMEM((cchunk, width), jnp.float32)
              for _ in range(grp)]
  scratch += [pltpu.SemaphoreType.DMA for _ in range(2 * grp)]
  scratch.append(pltpu.VMEM_SHARED((_NP, width), jnp.float32))
  if with_deg:
    out_type.append(jax.ShapeDtypeStruct((_NC, _NP), jnp.float32))
    scratch += [
        pltpu.VMEM((cchunk,), jnp.float32),
        pltpu.VMEM((_RP,), jnp.float32),
        pltpu.VMEM_SHARED((_NP,), jnp.float32),
    ]
    scratch += [pltpu.SemaphoreType.DMA for _ in range(grp)]

  def body(table, src_hbm, dst_hbm, *refs):
    out_agg = refs[0]
    refs = refs[1:]
    if with_deg:
      out_deg, refs = refs[0], refs[1:]
    src_v, dst_v = refs[0], refs[1]
    rows = refs[2:2 + grp]
    gsem = refs[2 + grp:2 + 2 * grp]
    ssem = refs[2 + 2 * grp:2 + 3 * grp]
    acc_sh = refs[2 + 3 * grp]
    if with_deg:
      ones_v, zdeg_v, deg_sh = refs[3 + 3 * grp:6 + 3 * grp]
      dsem = refs[6 + 3 * grp:6 + 4 * grp]
    c = lax.axis_index("c")
    s = lax.axis_index("s")
    wid = c * _NS + s
    base = s * _RP

    npack = width // 16

    def zrow(t, carry):
      rows[0][t // npack, pl.ds((t % npack) * 16, 16)] = jnp.zeros(
          (16,), jnp.float32)
      return carry

    lax.fori_loop(0, cchunk * npack, zrow, 0)

    nfull = _RP // cchunk
    rem = _RP - nfull * cchunk
    for k in range(nfull):
      pltpu.sync_copy(rows[0], acc_sh.at[pl.ds(base + k * cchunk, cchunk)])
    if rem:
      pltpu.sync_copy(rows[0].at[pl.ds(0, rem)],
                      acc_sh.at[pl.ds(base + nfull * cchunk, rem)])

    if with_deg:
      def zdeg(t, carry):
        zdeg_v[pl.ds(t * 16, 16)] = jnp.zeros((16,), jnp.float32)
        return carry

      lax.fori_loop(0, _RP // 16, zdeg, 0)
      pltpu.sync_copy(zdeg_v, deg_sh.at[pl.ds(base, _RP)])

      def ones(t, carry):
        ones_v[pl.ds(t * 16, 16)] = jnp.ones((16,), jnp.float32)
        return carry

      lax.fori_loop(0, cchunk // 16, ones, 0)

    pltpu.sync_copy(src_hbm.at[wid], src_v)
    pltpu.sync_copy(dst_hbm.at[wid], dst_v)

    plsc.subcore_barrier()

    if grp == 1:
      def chunk(j, carry):
        pltpu.async_copy(table.at[src_v.at[j]], rows[0], gsem[0]).wait()
        pltpu.sync_copy(rows[0], acc_sh.at[dst_v.at[j]], add=True)
        if with_deg:
          pltpu.sync_copy(ones_v, deg_sh.at[dst_v.at[j]], add=True)
        return carry

      lax.fori_loop(0, kchunks, chunk, 0)
    else:
      def group(g, carry):
        ch0 = g * grp
        gd = [pltpu.async_copy(table.at[src_v.at[ch0 + t]], rows[t],
                               gsem[t]) for t in range(grp)]
        sd = []
        dd = []
        for t in range(grp):
          gd[t].wait()
          sd.append(pltpu.async_copy(rows[t],
                                     acc_sh.at[dst_v.at[ch0 + t]],
                                     ssem[t], add=True))
          if with_deg:
            dd.append(pltpu.async_copy(ones_v,
                                       deg_sh.at[dst_v.at[ch0 + t]],
                                       dsem[t], add=True))
        for t in range(grp):
          sd[t].wait()
          if with_deg:
            dd[t].wait()
        return carry

      lax.fori_loop(0, kchunks // grp, group, 0)

    plsc.subcore_barrier()

    for k in range(_RP // 128):
      sl = pl.ds(base + k * 128, 128)
      pltpu.sync_copy(acc_sh.at[sl], out_agg.at[c, sl])
    if with_deg:
      pltpu.sync_copy(deg_sh.at[pl.ds(base, _RP)],
                      out_deg.at[c, pl.ds(base, _RP)])

  return pl.kernel(body, out_type=tuple(out_type), mesh=mesh,
                   scratch_types=scratch,
                   compiler_params=pltpu.CompilerParams(
                       use_tc_tiling_on_sc=False))


_sc_pass1 = _make_sc_pass(_D, _C1, _K1, _G1, True)
_sc_pass2 = _make_sc_pass(_O, _C2, _K2, _G2, False)

_BR = 1024
_GRID = _NP // _BR


def _tc_a_body(x_ref, ws1, wn1, sb, cb, agg, degt, wn2, h_ref, hw2_ref):
  d = jnp.maximum(degt[:, 0:1] + degt[:, 1:2], 1.0)
  hn = (agg[0] + agg[1]) / d
  hl = jnp.dot(x_ref[...], ws1[...], preferred_element_type=jnp.float32)
  hl = hl + jnp.dot(hn, wn1[...], preferred_element_type=jnp.float32)
  h = jnp.maximum(hl * sb[...] + cb[...], 0.0)
  h_ref[...] = h
  hw2_ref[...] = jnp.dot(h, wn2[...], preferred_element_type=jnp.float32)


_tc_a = pl.pallas_call(
    _tc_a_body,
    grid=(_GRID,),
    in_specs=[
        pl.BlockSpec((_BR, _D), lambda i: (i, 0)),
        pl.BlockSpec((_D, _H), lambda i: (0, 0)),
        pl.BlockSpec((_D, _H), lambda i: (0, 0)),
        pl.BlockSpec((1, _H), lambda i: (0, 0)),
        pl.BlockSpec((1, _H), lambda i: (0, 0)),
        pl.BlockSpec((_NC, _BR, _D), lambda i: (0, i, 0)),
        pl.BlockSpec((_BR, _NC), lambda i: (i, 0)),
        pl.BlockSpec((_H, _O), lambda i: (0, 0)),
    ],
    out_specs=[
        pl.BlockSpec((_BR, _H), lambda i: (i, 0)),
        pl.BlockSpec((_BR, _O), lambda i: (i, 0)),
    ],
    out_shape=[
        jax.ShapeDtypeStruct((_N, _H), jnp.float32),
        jax.ShapeDtypeStruct((_N, _O), jnp.float32),
    ],
)


def _tc_b_body(h_ref, ws2, agg2, degt, b2, out_ref):
  d = jnp.maximum(degt[:, 0:1] + degt[:, 1:2], 1.0)
  hn2 = (agg2[0] + agg2[1]) / d
  out_ref[...] = (
      jnp.dot(h_ref[...], ws2[...], preferred_element_type=jnp.float32)
      + hn2 + b2[...])


_tc_b = pl.pallas_call(
    _tc_b_body,
    grid=(_GRID,),
    in_specs=[
        pl.BlockSpec((_BR, _H), lambda i: (i, 0)),
        pl.BlockSpec((_H, _O), lambda i: (0, 0)),
        pl.BlockSpec((_NC, _BR, _O), lambda i: (0, i, 0)),
        pl.BlockSpec((_BR, _NC), lambda i: (i, 0)),
        pl.BlockSpec((1, _O), lambda i: (0, 0)),
    ],
    out_specs=pl.BlockSpec((_BR, _O), lambda i: (i, 0)),
    out_shape=jax.ShapeDtypeStruct((_N, _O), jnp.float32),
)


def kernel(x, edge_index, W_self1, W_neigh1, b1, gamma1, beta1,
           W_self2, W_neigh2, b2):
  def pad_edges(ep, k, cc):
    pad = ep - _E
    srcp = jnp.concatenate([edge_index[0], jnp.zeros((pad,), jnp.int32)])
    pad_dst = _N + jnp.arange(pad, dtype=jnp.int32) % (_NP - _N)
    dstp = jnp.concatenate([edge_index[1], pad_dst])
    return srcp.reshape(_NW, k, cc), dstp.reshape(_NW, k, cc)

  s1, d1 = pad_edges(_EP1, _K1, _C1)
  s2, d2 = pad_edges(_EP2, _K2, _C2)

  aggx, deg = _sc_pass1(x, s1, d1)
  degt = deg.T

  sb = (gamma1 * (1.0 / jnp.sqrt(1.0 + 1e-5))).reshape(1, _H)
  cb = (b1 * sb[0] + beta1).reshape(1, _H)

  h, hw2 = _tc_a(x, W_self1, W_neigh1, sb, cb, aggx, degt, W_neigh2)
  (agg2,) = _sc_pass2(hw2, s2, d2)
  out = _tc_b(h, W_self2, agg2, degt, b2.reshape(1, _O))
  return out


# speedup vs baseline: 2.8586x; 1.1801x over previous
import functools

import jax
import jax.numpy as jnp
from jax import lax
from jax.experimental import pallas as pl
from jax.experimental.pallas import tpu as pltpu
from jax.experimental.pallas import tpu_sc as plsc

_N = 10000
_E = 320000
_D = 128
_H = 128
_O = 64

_NC = 2
_NS = 16
_NW = _NC * _NS
_NP = 10240
_RP = _NP // _NS

_C1, _K1, _G1 = 80, 125, 1
_C2, _K2, _G2 = 80, 125, 5
_EP1 = _NW * _K1 * _C1
_EP2 = _NW * _K2 * _C2


def _make_sc_pass(width, cchunk, kchunks, grp, with_deg):
  mesh = plsc.VectorSubcoreMesh(core_axis_name="c", subcore_axis_name="s")
  out_type = [jax.ShapeDtypeStruct((_NC, _NP, width), jnp.float32)]
  scratch = [
      pltpu.VMEM((kchunks, cchunk), jnp.int32),
      pltpu.VMEM((kchunks, cchunk), jnp.int32),
  ]
  scratch += [pltpu.VMEM((cchunk, width), jnp.float32)
              for _ in range(grp)]
  scratch += [pltpu.SemaphoreType.DMA for _ in range(2 * grp)]
  scratch.append(pltpu.VMEM_SHARED((_NP, width), jnp.float32))
  if with_deg:
    out_type.append(jax.ShapeDtypeStruct((_NC, _NP), jnp.float32))
    scratch += [
        pltpu.VMEM((cchunk,), jnp.float32),
        pltpu.VMEM((_RP,), jnp.float32),
        pltpu.VMEM_SHARED((_NP,), jnp.float32),
    ]
    scratch += [pltpu.SemaphoreType.DMA for _ in range(grp)]

  def body(table, src_hbm, dst_hbm, *refs):
    out_agg = refs[0]
    refs = refs[1:]
    if with_deg:
      out_deg, refs = refs[0], refs[1:]
    src_v, dst_v = refs[0], refs[1]
    rows = refs[2:2 + grp]
    gsem = refs[2 + grp:2 + 2 * grp]
    ssem = refs[2 + 2 * grp:2 + 3 * grp]
    acc_sh = refs[2 + 3 * grp]
    if with_deg:
      ones_v, zdeg_v, deg_sh = refs[3 + 3 * grp:6 + 3 * grp]
      dsem = refs[6 + 3 * grp:6 + 4 * grp]
    c = lax.axis_index("c")
    s = lax.axis_index("s")
    wid = c * _NS + s
    base = s * _RP

    npack = width // 16

    def zrow(t, carry):
      rows[0][t // npack, pl.ds((t % npack) * 16, 16)] = jnp.zeros(
          (16,), jnp.float32)
      return carry

    lax.fori_loop(0, cchunk * npack, zrow, 0)

    nfull = _RP // cchunk
    rem = _RP - nfull * cchunk
    for k in range(nfull):
      pltpu.sync_copy(rows[0], acc_sh.at[pl.ds(base + k * cchunk, cchunk)])
    if rem:
      pltpu.sync_copy(rows[0].at[pl.ds(0, rem)],
                      acc_sh.at[pl.ds(base + nfull * cchunk, rem)])

    if with_deg:
      def zdeg(t, carry):
        zdeg_v[pl.ds(t * 16, 16)] = jnp.zeros((16,), jnp.float32)
        return carry

      lax.fori_loop(0, _RP // 16, zdeg, 0)
      pltpu.sync_copy(zdeg_v, deg_sh.at[pl.ds(base, _RP)])

      def ones(t, carry):
        ones_v[pl.ds(t * 16, 16)] = jnp.ones((16,), jnp.float32)
        return carry

      lax.fori_loop(0, cchunk // 16, ones, 0)

    pltpu.sync_copy(src_hbm.at[wid], src_v)
    pltpu.sync_copy(dst_hbm.at[wid], dst_v)

    plsc.subcore_barrier()

    if grp == 1:
      def chunk(j, carry):
        pltpu.async_copy(table.at[src_v.at[j]], rows[0], gsem[0]).wait()
        pltpu.sync_copy(rows[0], acc_sh.at[dst_v.at[j]], add=True)
        if with_deg:
          pltpu.sync_copy(ones_v, deg_sh.at[dst_v.at[j]], add=True)
        return carry

      lax.fori_loop(0, kchunks, chunk, 0)
    else:
      def group(g, carry):
        ch0 = g * grp
        gd = [pltpu.async_copy(table.at[src_v.at[ch0 + t]], rows[t],
                               gsem[t]) for t in range(grp)]
        sd = []
        dd = []
        for t in range(grp):
          gd[t].wait()
          sd.append(pltpu.async_copy(rows[t],
                                     acc_sh.at[dst_v.at[ch0 + t]],
                                     ssem[t], add=True))
          if with_deg:
            dd.append(pltpu.async_copy(ones_v,
                                       deg_sh.at[dst_v.at[ch0 + t]],
                                       dsem[t], add=True))
        for t in range(grp):
          sd[t].wait()
          if with_deg:
            dd[t].wait()
        return carry

      lax.fori_loop(0, kchunks // grp, group, 0)

    plsc.subcore_barrier()

    for k in range(_RP // 128):
      sl = pl.ds(base + k * 128, 128)
      pltpu.sync_copy(acc_sh.at[sl], out_agg.at[c, sl])
    if with_deg:
      pltpu.sync_copy(deg_sh.at[pl.ds(base, _RP)],
                      out_deg.at[c, pl.ds(base, _RP)])

  return pl.kernel(body, out_type=tuple(out_type), mesh=mesh,
                   scratch_types=scratch,
                   compiler_params=pltpu.CompilerParams(
                       use_tc_tiling_on_sc=False))


_sc_pass1 = _make_sc_pass(_D, _C1, _K1, _G1, True)
_sc_pass2 = _make_sc_pass(_O, _C2, _K2, _G2, False)

_BR = 1024
_GRID = _NP // _BR


def _tc_a_body(x_ref, ws1, wn1, sb, cb, agg, degt, wn2, h_ref, hw2_ref):
  d = jnp.maximum(degt[:, 0:1] + degt[:, 1:2], 1.0)
  hn = (agg[0] + agg[1]) / d
  hl = jnp.dot(x_ref[...], ws1[...], preferred_element_type=jnp.float32)
  hl = hl + jnp.dot(hn, wn1[...], preferred_element_type=jnp.float32)
  h = jnp.maximum(hl * sb[...] + cb[...], 0.0)
  h_ref[...] = h
  hw2_ref[...] = jnp.dot(h, wn2[...], preferred_element_type=jnp.float32)


_tc_a = pl.pallas_call(
    _tc_a_body,
    grid=(_GRID,),
    in_specs=[
        pl.BlockSpec((_BR, _D), lambda i: (i, 0)),
        pl.BlockSpec((_D, _H), lambda i: (0, 0)),
        pl.BlockSpec((_D, _H), lambda i: (0, 0)),
        pl.BlockSpec((1, _H), lambda i: (0, 0)),
        pl.BlockSpec((1, _H), lambda i: (0, 0)),
        pl.BlockSpec((_NC, _BR, _D), lambda i: (0, i, 0)),
        pl.BlockSpec((_BR, _NC), lambda i: (i, 0)),
        pl.BlockSpec((_H, _O), lambda i: (0, 0)),
    ],
    out_specs=[
        pl.BlockSpec((_BR, _H), lambda i: (i, 0)),
        pl.BlockSpec((_BR, _O), lambda i: (i, 0)),
    ],
    out_shape=[
        jax.ShapeDtypeStruct((_N, _H), jnp.float32),
        jax.ShapeDtypeStruct((_N, _O), jnp.float32),
    ],
)


def _tc_b_body(h_ref, ws2, agg2, degt, b2, out_ref):
  d = jnp.maximum(degt[:, 0:1] + degt[:, 1:2], 1.0)
  hn2 = (agg2[0] + agg2[1]) / d
  out_ref[...] = (
      jnp.dot(h_ref[...], ws2[...], preferred_element_type=jnp.float32)
      + hn2 + b2[...])


_tc_b = pl.pallas_call(
    _tc_b_body,
    grid=(_GRID,),
    in_specs=[
        pl.BlockSpec((_BR, _H), lambda i: (i, 0)),
        pl.BlockSpec((_H, _O), lambda i: (0, 0)),
        pl.BlockSpec((_NC, _BR, _O), lambda i: (0, i, 0)),
        pl.BlockSpec((_BR, _NC), lambda i: (i, 0)),
        pl.BlockSpec((1, _O), lambda i: (0, 0)),
    ],
    out_specs=pl.BlockSpec((_BR, _O), lambda i: (i, 0)),
    out_shape=jax.ShapeDtypeStruct((_N, _O), jnp.float32),
)


def kernel(x, edge_index, W_self1, W_neigh1, b1, gamma1, beta1,
           W_self2, W_neigh2, b2):
  def pad_edges(ep, k, cc):
    pad = ep - _E
    srcp = jnp.concatenate([edge_index[0], jnp.zeros((pad,), jnp.int32)])
    pad_dst = _N + jnp.arange(pad, dtype=jnp.int32) % (_NP - _N)
    dstp = jnp.concatenate([edge_index[1], pad_dst])
    return srcp.reshape(_NW, k, cc), dstp.reshape(_NW, k, cc)

  s1, d1 = pad_edges(_EP1, _K1, _C1)
  s2, d2 = pad_edges(_EP2, _K2, _C2)

  aggx, deg = _sc_pass1(x, s1, d1)
  degt = deg.T

  sb = (gamma1 * (1.0 / jnp.sqrt(1.0 + 1e-5))).reshape(1, _H)
  cb = (b1 * sb[0] + beta1).reshape(1, _H)

  h, hw2 = _tc_a(x, W_self1, W_neigh1, sb, cb, aggx, degt, W_neigh2)
  (agg2,) = _sc_pass2(hw2, s2, d2)
  out = _tc_b(h, W_self2, agg2, degt, b2.reshape(1, _O))
  return out
